# baseline (speedup 1.0000x reference)
import functools

import jax
import jax.numpy as jnp
from jax import lax
from jax.experimental import pallas as pl
from jax.experimental.pallas import tpu as pltpu
from jax.experimental.pallas import tpu_sc as plsc

CONTEXT_INFLUENCE = 0.5
GRAPH_BIAS_SCALE = 0.8
COMMITMENT_COST = 0.25


def _prep_body(a_ref, cb_ref, o_ref, cn_ref):
    o_ref[:] = (GRAPH_BIAS_SCALE * CONTEXT_INFLUENCE) * jax.nn.sigmoid(a_ref[:])
    cb = cb_ref[:]
    cn_ref[0, :] = jnp.sum(cb * cb, axis=1)


def _prep_tables(adjacency, codebook):
    K = adjacency.shape[0]
    return pl.pallas_call(
        _prep_body,
        out_shape=[
            jax.ShapeDtypeStruct((K, K), jnp.float32),
            jax.ShapeDtypeStruct((1, K), jnp.float32),
        ],
    )(adjacency, codebook)


@functools.lru_cache(maxsize=None)
def _make_sc_gather(V, D, Btok, CH):
    info = plsc.get_sparse_core_info()
    NW = info.num_cores * info.num_subcores
    b_per_w = Btok // NW
    n_ch = b_per_w // CH
    mesh = plsc.VectorSubcoreMesh(core_axis_name="c", subcore_axis_name="s")

    @functools.partial(
        pl.kernel,
        mesh=mesh,
        out_type=jax.ShapeDtypeStruct((Btok, D), jnp.float32),
        scratch_types=[
            pltpu.VMEM((b_per_w,), jnp.int32),
            pltpu.VMEM((CH, D), jnp.float32),
            pltpu.VMEM((CH, D), jnp.float32),
            pltpu.SemaphoreType.DMA,
            pltpu.SemaphoreType.DMA,
            pltpu.SemaphoreType.DMA,
            pltpu.SemaphoreType.DMA,
        ],
    )
    def sc_gather(table_hbm, idx_hbm, out_hbm, idx_v, buf0, buf1,
                  gs0, gs1, ws0, ws1):
        wid = lax.axis_index("s") * info.num_cores + lax.axis_index("c")
        base = wid * b_per_w
        pltpu.sync_copy(idx_hbm.at[pl.ds(base, b_per_w)], idx_v)
        bufs, gsems, wsems = (buf0, buf1), (gs0, gs1), (ws0, ws1)

        def g_copy(i, b):
            return pltpu.make_async_copy(
                table_hbm.at[idx_v.at[pl.ds(i * CH, CH)]], bufs[b], gsems[b])

        def w_copy(i, b):
            return pltpu.make_async_copy(
                bufs[b], out_hbm.at[pl.ds(base + i * CH, CH)], wsems[b])

        g_copy(0, 0).start()

        def body(g, _):
            for b in range(2):
                i = g * 2 + b
                g_copy(i, b).wait()
                w_copy(i, b).start()
                ob = 1 - b

                @pl.when(i + 1 < n_ch)
                def _():
                    @pl.when(i >= 1)
                    def _():
                        w_copy(i - 1, ob).wait()

                    g_copy(i + 1, ob).start()

            return 0

        lax.fori_loop(0, n_ch // 2, body, 0)
        w_copy(n_ch - 2, 0).wait()
        w_copy(n_ch - 1, 1).wait()

    return sc_gather


def _vq_body(zr_ref, zi_ref, hr_ref, hi_ref, prior_ref, cb_ref, gw_ref,
             gb_ref, cn_ref, idx_ref, zq_ref, loss_ref):
    K = cb_ref.shape[0]
    z = jnp.concatenate([zr_ref[:], zi_ref[:]], axis=1)
    ctx = jnp.concatenate([hr_ref[:], hi_ref[:]], axis=1)
    cb = cb_ref[:]
    t = lax.dot_general(z, cb, (((1,), (1,)), ((), ())))
    ctxb = jnp.dot(ctx, gw_ref[:]) + gb_ref[0, :][None, :]
    d = cn_ref[0, :][None, :] - 2.0 * t - prior_ref[:] * (2.0 + jnp.tanh(ctxb))
    m = jnp.min(d, axis=1, keepdims=True)
    colv = lax.broadcasted_iota(jnp.int32, d.shape, 1)
    idx = jnp.min(jnp.where(d == m, colv, K), axis=1)
    idx_ref[0, 0, :] = idx
    oh = (colv == idx[:, None]).astype(jnp.float32)
    zq = lax.dot_general(oh, cb, (((1,), (0,)), ((), ())))
    zq_ref[:] = zq
    diff = zq - z
    part = jnp.sum(diff * diff).reshape(1, 1)

    @pl.when(pl.program_id(0) == 0)
    def _init():
        loss_ref[:, :] = jnp.zeros_like(part)

    loss_ref[:, :] += part


def _vq_call(zr, zi, hr, hi, prior, codebook, gate_W, gb2, cnorm2, BB, Bs, off):
    _, L = zr.shape
    K, D = codebook.shape
    NB = Bs // BB
    return pl.pallas_call(
        _vq_body,
        grid=(NB,),
        in_specs=[
            pl.BlockSpec((BB, L), lambda i, o=off: (i + o, 0)),
            pl.BlockSpec((BB, L), lambda i, o=off: (i + o, 0)),
            pl.BlockSpec((BB, L), lambda i, o=off: (i + o, 0)),
            pl.BlockSpec((BB, L), lambda i, o=off: (i + o, 0)),
            pl.BlockSpec((BB, K), lambda i: (i, 0)),
            pl.BlockSpec((K, D), lambda i: (0, 0)),
            pl.BlockSpec((D, K), lambda i: (0, 0)),
            pl.BlockSpec((1, K), lambda i: (0, 0)),
            pl.BlockSpec((1, K), lambda i: (0, 0)),
        ],
        out_specs=[
            pl.BlockSpec((1, 1, BB), lambda i: (i, 0, 0)),
            pl.BlockSpec((BB, D), lambda i: (i, 0)),
            pl.BlockSpec((1, 1), lambda i: (0, 0)),
        ],
        out_shape=[
            jax.ShapeDtypeStruct((NB, 1, BB), jnp.int32),
            jax.ShapeDtypeStruct((Bs, D), jnp.float32),
            jax.ShapeDtypeStruct((1, 1), jnp.float32),
        ],
    )(zr, zi, hr, hi, prior, codebook, gate_W, gb2, cnorm2)


def kernel(z_real, z_imag, h_real, h_imag, prev_symbol_idx, codebook,
           adjacency, gate_W, gate_b):
    B, L = z_real.shape
    K, D = codebook.shape
    BB = 1024
    S = 4
    Bs = B // S

    sigA, cnorm2 = _prep_tables(adjacency, codebook)
    gb2 = gate_b.reshape(1, K)
    idx32 = prev_symbol_idx.astype(jnp.int32)
    gather = _make_sc_gather(K, K, Bs, 32)

    parts = []
    for s in range(S):
        prior = gather(sigA, lax.dynamic_slice_in_dim(idx32, s * Bs, Bs))
        parts.append(_vq_call(z_real, z_imag, h_real, h_imag, prior,
                              codebook, gate_W, gb2, cnorm2, BB, Bs,
                              s * (Bs // BB)))

    min_indices = jnp.concatenate([p[0].reshape(Bs) for p in parts])
    zq = jnp.concatenate([p[1] for p in parts])
    loss_sum = sum(p[2][0, 0] for p in parts)
    loss = (1.0 + COMMITMENT_COST) / (B * D) * loss_sum
    z_complex = lax.complex(zq[:, :L], zq[:, L:])
    return z_complex, loss, min_indices


# speedup vs baseline: 1.0148x; 1.0148x over previous
import functools

import jax
import jax.numpy as jnp
from jax import lax
from jax.experimental import pallas as pl
from jax.experimental.pallas import tpu as pltpu
from jax.experimental.pallas import tpu_sc as plsc

CONTEXT_INFLUENCE = 0.5
GRAPH_BIAS_SCALE = 0.8
COMMITMENT_COST = 0.25


def _prep_body(a_ref, cb_ref, o_ref, cn_ref):
    o_ref[:] = (GRAPH_BIAS_SCALE * CONTEXT_INFLUENCE) * jax.nn.sigmoid(a_ref[:])
    cb = cb_ref[:]
    cn_ref[0, :] = jnp.sum(cb * cb, axis=1)


def _prep_tables(adjacency, codebook):
    K = adjacency.shape[0]
    return pl.pallas_call(
        _prep_body,
        out_shape=[
            jax.ShapeDtypeStruct((K, K), jnp.float32),
            jax.ShapeDtypeStruct((1, K), jnp.float32),
        ],
    )(adjacency, codebook)


@functools.lru_cache(maxsize=None)
def _make_sc_gather(V, D, Btok, CH):
    info = plsc.get_sparse_core_info()
    NW = info.num_cores * info.num_subcores
    b_per_w = Btok // NW
    n_ch = b_per_w // CH
    mesh = plsc.VectorSubcoreMesh(core_axis_name="c", subcore_axis_name="s")

    @functools.partial(
        pl.kernel,
        mesh=mesh,
        out_type=jax.ShapeDtypeStruct((Btok, D), jnp.float32),
        scratch_types=[
            pltpu.VMEM((b_per_w,), jnp.int32),
            pltpu.VMEM((CH, D), jnp.float32),
            pltpu.VMEM((CH, D), jnp.float32),
            pltpu.SemaphoreType.DMA,
            pltpu.SemaphoreType.DMA,
            pltpu.SemaphoreType.DMA,
            pltpu.SemaphoreType.DMA,
        ],
    )
    def sc_gather(table_hbm, idx_hbm, out_hbm, idx_v, buf0, buf1,
                  gs0, gs1, ws0, ws1):
        wid = lax.axis_index("s") * info.num_cores + lax.axis_index("c")
        base = wid * b_per_w
        pltpu.sync_copy(idx_hbm.at[pl.ds(base, b_per_w)], idx_v)
        bufs, gsems, wsems = (buf0, buf1), (gs0, gs1), (ws0, ws1)

        def g_copy(i, b):
            return pltpu.make_async_copy(
                table_hbm.at[idx_v.at[pl.ds(i * CH, CH)]], bufs[b], gsems[b])

        def w_copy(i, b):
            return pltpu.make_async_copy(
                bufs[b], out_hbm.at[pl.ds(base + i * CH, CH)], wsems[b])

        g_copy(0, 0).start()

        def body(g, _):
            for b in range(2):
                i = g * 2 + b
                g_copy(i, b).wait()
                w_copy(i, b).start()
                ob = 1 - b

                @pl.when(i + 1 < n_ch)
                def _():
                    @pl.when(i >= 1)
                    def _():
                        w_copy(i - 1, ob).wait()

                    g_copy(i + 1, ob).start()

            return 0

        lax.fori_loop(0, n_ch // 2, body, 0)
        w_copy(n_ch - 2, 0).wait()
        w_copy(n_ch - 1, 1).wait()

    return sc_gather


def _vq_body(zr_ref, zi_ref, hr_ref, hi_ref, prior_ref, cb_ref, gw_ref,
             gb_ref, cn_ref, idx_ref, zq_ref, loss_ref):
    K = cb_ref.shape[0]
    z = jnp.concatenate([zr_ref[:], zi_ref[:]], axis=1)
    ctx = jnp.concatenate([hr_ref[:], hi_ref[:]], axis=1)
    cb = cb_ref[:]
    t = lax.dot_general(z, cb, (((1,), (1,)), ((), ())))
    ctxb = jnp.dot(ctx, gw_ref[:]) + gb_ref[0, :][None, :]
    d = cn_ref[0, :][None, :] - 2.0 * t - prior_ref[:] * (2.0 + jnp.tanh(ctxb))
    m = jnp.min(d, axis=1, keepdims=True)
    colv = lax.broadcasted_iota(jnp.int32, d.shape, 1)
    idx = jnp.min(jnp.where(d == m, colv, K), axis=1)
    idx_ref[0, 0, :] = idx
    oh = (colv == idx[:, None]).astype(jnp.float32)
    zq = lax.dot_general(oh, cb, (((1,), (0,)), ((), ())))
    zq_ref[:] = zq
    diff = zq - z
    part = jnp.sum(diff * diff).reshape(1, 1)

    @pl.when(pl.program_id(0) == 0)
    def _init():
        loss_ref[:, :] = jnp.zeros_like(part)

    loss_ref[:, :] += part


def kernel(z_real, z_imag, h_real, h_imag, prev_symbol_idx, codebook,
           adjacency, gate_W, gate_b):
    B, L = z_real.shape
    K, D = codebook.shape
    BB = 1024
    NB = B // BB

    sigA, cnorm2 = _prep_tables(adjacency, codebook)
    prior = _make_sc_gather(K, K, B, 32)(sigA, prev_symbol_idx.astype(jnp.int32))

    gb2 = gate_b.reshape(1, K)
    idx2, zq, loss_acc = pl.pallas_call(
        _vq_body,
        grid=(NB,),
        in_specs=[
            pl.BlockSpec((BB, L), lambda i: (i, 0)),
            pl.BlockSpec((BB, L), lambda i: (i, 0)),
            pl.BlockSpec((BB, L), lambda i: (i, 0)),
            pl.BlockSpec((BB, L), lambda i: (i, 0)),
            pl.BlockSpec((BB, K), lambda i: (i, 0)),
            pl.BlockSpec((K, D), lambda i: (0, 0)),
            pl.BlockSpec((D, K), lambda i: (0, 0)),
            pl.BlockSpec((1, K), lambda i: (0, 0)),
            pl.BlockSpec((1, K), lambda i: (0, 0)),
        ],
        out_specs=[
            pl.BlockSpec((1, 1, BB), lambda i: (i, 0, 0)),
            pl.BlockSpec((BB, D), lambda i: (i, 0)),
            pl.BlockSpec((1, 1), lambda i: (0, 0)),
        ],
        out_shape=[
            jax.ShapeDtypeStruct((NB, 1, BB), jnp.int32),
            jax.ShapeDtypeStruct((B, D), jnp.float32),
            jax.ShapeDtypeStruct((1, 1), jnp.float32),
        ],
    )(z_real, z_imag, h_real, h_imag, prior, codebook, gate_W, gb2, cnorm2)

    min_indices = idx2.reshape(B)
    loss = (1.0 + COMMITMENT_COST) / (B * D) * loss_acc[0, 0]
    z_complex = lax.complex(zq[:, :L], zq[:, L:])
    return z_complex, loss, min_indices
